# baseline (speedup 1.0000x reference)
import jax
import jax.numpy as jnp
from jax import lax
from jax.experimental import pallas as pl
from jax.experimental.pallas import tpu as pltpu


def _gate_pool_kernel(x_ref, padm_ref, ids_ref, gW1_ref, gb1_ref, gW2_ref,
                      gb2_ref, e_ref, s_ref, praw_ref):
    x = x_ref[0]
    h = jnp.tanh(jnp.dot(x, gW1_ref[...],
                         preferred_element_type=jnp.float32) + gb1_ref[...])
    g = jnp.sum(h * gW2_ref[...], axis=1, keepdims=True) + gb2_ref[...]
    e = jnp.exp(g) * padm_ref[0]
    e_ref[0] = e
    n = x.shape[0]
    c = s_ref.shape[-1]
    onehot = lax.broadcasted_iota(jnp.int32, (n, c), 1) == ids_ref[0]
    numer = jnp.where(onehot, e, 0.0)
    s_ref[0] = jnp.sum(numer, axis=0, keepdims=True)
    praw_ref[0] = lax.dot_general(numer, x, (((0,), (0,)), ((), ())),
                                  preferred_element_type=jnp.float32)


def _proj_kernel(praw_ref, st_ref, embT_ref, ct_ref, pW_ref, pb_ref,
                 lng_ref, lnb_ref, out_ref):
    b = pl.program_id(0)
    c, nb = st_ref.shape
    t = embT_ref.shape[1]
    onehot_b = (lax.broadcasted_iota(jnp.int32, (nb, 1), 0) == b
                ).astype(jnp.float32)
    s_col = jnp.dot(st_ref[...], onehot_b,
                    preferred_element_type=jnp.float32)
    ct_b = ct_ref[b]
    onehot_t = (lax.broadcasted_iota(jnp.int32, (t, 1), 0) == ct_b
                ).astype(jnp.float32)
    ctm_col = jnp.dot(embT_ref[...], onehot_t,
                      preferred_element_type=jnp.float32)
    any_col = jnp.sum(st_ref[...], axis=1, keepdims=True) > 0.0
    nonempty = s_col > 0.0
    scale = jnp.where(nonempty,
                      (1.0 + 0.1 * ctm_col) / jnp.where(nonempty, s_col, 1.0),
                      0.0)
    pooled = praw_ref[0] * scale
    proj = jnp.dot(pooled, pW_ref[...],
                   preferred_element_type=jnp.float32) + pb_ref[...]
    mean = jnp.mean(proj, axis=1, keepdims=True)
    d = proj - mean
    var = jnp.mean(d * d, axis=1, keepdims=True)
    y = d * lax.rsqrt(var + 1e-5) * lng_ref[...] + lnb_ref[...]
    y = jnp.where(y > 0.0, y, jnp.exp(jnp.minimum(y, 0.0)) - 1.0)
    out_ref[0] = jnp.where(any_col, y, 0.0)


def kernel(x, gW1, gb1, gW2, gb2, emb, pW, pb, ln_g, ln_b, cancer_type,
           channel_ids, pad_mask):
    B, N, H = x.shape
    T, C = emb.shape
    padm = (~pad_mask).astype(jnp.float32).reshape(B, N, 1)
    ids3 = channel_ids.astype(jnp.int32).reshape(B, N, 1)

    e, S, praw = pl.pallas_call(
        _gate_pool_kernel,
        grid=(B,),
        in_specs=[
            pl.BlockSpec((1, N, H), lambda b: (b, 0, 0)),
            pl.BlockSpec((1, N, 1), lambda b: (b, 0, 0)),
            pl.BlockSpec((1, N, 1), lambda b: (b, 0, 0)),
            pl.BlockSpec((H, H // 2), lambda b: (0, 0)),
            pl.BlockSpec((1, H // 2), lambda b: (0, 0)),
            pl.BlockSpec((1, H // 2), lambda b: (0, 0)),
            pl.BlockSpec((1, 1), lambda b: (0, 0)),
        ],
        out_specs=[
            pl.BlockSpec((1, N, 1), lambda b: (b, 0, 0)),
            pl.BlockSpec((1, 1, C), lambda b: (b, 0, 0)),
            pl.BlockSpec((1, C, H), lambda b: (b, 0, 0)),
        ],
        out_shape=[
            jax.ShapeDtypeStruct((B, N, 1), jnp.float32),
            jax.ShapeDtypeStruct((B, 1, C), jnp.float32),
            jax.ShapeDtypeStruct((B, C, H), jnp.float32),
        ],
    )(x, padm, ids3, gW1, gb1.reshape(1, -1), gW2.reshape(1, -1),
      gb2.reshape(1, 1))

    Smat = S[:, 0, :]

    tokens = pl.pallas_call(
        _proj_kernel,
        grid=(B,),
        in_specs=[
            pl.BlockSpec((1, C, H), lambda b: (b, 0, 0)),
            pl.BlockSpec((C, B), lambda b: (0, 0)),
            pl.BlockSpec((C, T), lambda b: (0, 0)),
            pl.BlockSpec(memory_space=pltpu.SMEM),
            pl.BlockSpec((H, H), lambda b: (0, 0)),
            pl.BlockSpec((1, H), lambda b: (0, 0)),
            pl.BlockSpec((1, H), lambda b: (0, 0)),
            pl.BlockSpec((1, H), lambda b: (0, 0)),
        ],
        out_specs=pl.BlockSpec((1, C, H), lambda b: (b, 0, 0)),
        out_shape=jax.ShapeDtypeStruct((B, C, H), jnp.float32),
    )(praw, Smat.T, emb.T, cancer_type.astype(jnp.int32), pW,
      pb.reshape(1, -1), ln_g.reshape(1, -1), ln_b.reshape(1, -1))

    channel_active = Smat > 0.0
    return tokens, channel_active


# speedup vs baseline: 2.4492x; 2.4492x over previous
import jax
import jax.numpy as jnp
from jax import lax
from jax.experimental import pallas as pl
from jax.experimental.pallas import tpu as pltpu


def _gate_pool_kernel(x_ref, ids_ref, gW1_ref, gb1_ref, gW2_ref, gb2_ref,
                      e_ref, s_ref, praw_ref):
    x = x_ref[0]
    h = jnp.tanh(jnp.dot(x, gW1_ref[...],
                         preferred_element_type=jnp.float32) + gb1_ref[...])
    g_col = jnp.dot(h, gW2_ref[...],
                    preferred_element_type=jnp.float32)
    g_row = g_col.T + gb2_ref[...]
    e_row = jnp.exp(g_row)
    e_ref[0] = e_row
    n = x.shape[0]
    c = praw_ref.shape[1]
    onehot = lax.broadcasted_iota(jnp.int32, (c, n), 0) == ids_ref[0]
    numer = jnp.where(onehot, e_row, 0.0)
    s_ref[0] = jnp.sum(numer, axis=1, keepdims=True)
    praw_ref[0] = jnp.dot(numer, x,
                          preferred_element_type=jnp.float32)


def _proj_kernel(praw_ref, st_ref, embT_ref, ct_ref, pW_ref, pb_ref,
                 lng_ref, lnb_ref, out_ref):
    b = pl.program_id(0)
    c, nb = st_ref.shape
    t = embT_ref.shape[1]
    onehot_b = (lax.broadcasted_iota(jnp.int32, (nb, 1), 0) == b
                ).astype(jnp.float32)
    s_col = jnp.dot(st_ref[...], onehot_b,
                    preferred_element_type=jnp.float32)
    ct_b = ct_ref[b]
    onehot_t = (lax.broadcasted_iota(jnp.int32, (t, 1), 0) == ct_b
                ).astype(jnp.float32)
    ctm_col = jnp.dot(embT_ref[...], onehot_t,
                      preferred_element_type=jnp.float32)
    any_col = jnp.sum(st_ref[...], axis=1, keepdims=True) > 0.0
    nonempty = s_col > 0.0
    scale = jnp.where(nonempty,
                      (1.0 + 0.1 * ctm_col) / jnp.where(nonempty, s_col, 1.0),
                      0.0)
    pooled = praw_ref[0] * scale
    proj = jnp.dot(pooled, pW_ref[...],
                   preferred_element_type=jnp.float32) + pb_ref[...]
    mean = jnp.mean(proj, axis=1, keepdims=True)
    d = proj - mean
    var = jnp.mean(d * d, axis=1, keepdims=True)
    y = d * lax.rsqrt(var + 1e-5) * lng_ref[...] + lnb_ref[...]
    y = jnp.where(y > 0.0, y, jnp.exp(jnp.minimum(y, 0.0)) - 1.0)
    out_ref[0] = jnp.where(any_col, y, 0.0)


def kernel(x, gW1, gb1, gW2, gb2, emb, pW, pb, ln_g, ln_b, cancer_type,
           channel_ids, pad_mask):
    B, N, H = x.shape
    T, C = emb.shape
    ids_m = jnp.where(pad_mask, C, channel_ids.astype(jnp.int32))
    ids2 = ids_m.reshape(B, 1, N)

    e, S, praw = pl.pallas_call(
        _gate_pool_kernel,
        grid=(B,),
        in_specs=[
            pl.BlockSpec((1, N, H), lambda b: (b, 0, 0)),
            pl.BlockSpec((1, 1, N), lambda b: (b, 0, 0)),
            pl.BlockSpec((H, H // 2), lambda b: (0, 0)),
            pl.BlockSpec((1, H // 2), lambda b: (0, 0)),
            pl.BlockSpec((H // 2, 1), lambda b: (0, 0)),
            pl.BlockSpec((1, 1), lambda b: (0, 0)),
        ],
        out_specs=[
            pl.BlockSpec((1, 1, N), lambda b: (b, 0, 0)),
            pl.BlockSpec((1, C, 1), lambda b: (b, 0, 0)),
            pl.BlockSpec((1, C, H), lambda b: (b, 0, 0)),
        ],
        out_shape=[
            jax.ShapeDtypeStruct((B, 1, N), jnp.float32),
            jax.ShapeDtypeStruct((B, C, 1), jnp.float32),
            jax.ShapeDtypeStruct((B, C, H), jnp.float32),
        ],
    )(x, ids2, gW1, gb1.reshape(1, -1), gW2, gb2.reshape(1, 1))

    Smat = S[:, :, 0]

    tokens = pl.pallas_call(
        _proj_kernel,
        grid=(B,),
        in_specs=[
            pl.BlockSpec((1, C, H), lambda b: (b, 0, 0)),
            pl.BlockSpec((C, B), lambda b: (0, 0)),
            pl.BlockSpec((C, T), lambda b: (0, 0)),
            pl.BlockSpec(memory_space=pltpu.SMEM),
            pl.BlockSpec((H, H), lambda b: (0, 0)),
            pl.BlockSpec((1, H), lambda b: (0, 0)),
            pl.BlockSpec((1, H), lambda b: (0, 0)),
            pl.BlockSpec((1, H), lambda b: (0, 0)),
        ],
        out_specs=pl.BlockSpec((1, C, H), lambda b: (b, 0, 0)),
        out_shape=jax.ShapeDtypeStruct((B, C, H), jnp.float32),
    )(praw, Smat.T, emb.T, cancer_type.astype(jnp.int32), pW,
      pb.reshape(1, -1), ln_g.reshape(1, -1), ln_b.reshape(1, -1))

    channel_active = Smat > 0.0
    return tokens, channel_active
